# Initial kernel scaffold; baseline (speedup 1.0000x reference)
import jax
import jax.numpy as jnp
from jax.experimental import pallas as pl


def kernel(x, edge_index, W1, b1, g1, be1, W2, b2, g2, be2, Wg1, bg1, gg1, gb1, Wg2, bg2, gg2, gb2):
    raise NotImplementedError("write your pallas kernel here")


# speedup vs baseline: 27.2620x; 27.2620x over previous
import functools

import jax
import jax.numpy as jnp
from jax import lax
from jax.experimental import pallas as pl
from jax.experimental.pallas import tpu as pltpu
from jax.experimental.pallas import tpu_sc as plsc

N = 100000
E = 1600000
IDXW = 128
SUPR = 8
EPAD = 1605632
NROWS = EPAD // IDXW
PADE = EPAD - E
ANODES = N + 8
RB = 2000
GRID = N // RB

def _mesh():
    return plsc.VectorSubcoreMesh(core_axis_name="c", subcore_axis_name="s")

ANODES1 = 16 * 6272
DEG_CH = 6272
ACC_CH = 6256
ACC_LAST = N - 15 * ACC_CH


def _sc_degree(dstp, zeros1d):
    @functools.partial(
        pl.kernel,
        out_type=(jax.ShapeDtypeStruct((ANODES1,), jnp.float32),
                  jax.ShapeDtypeStruct((ANODES1,), jnp.float32)),
        scratch_types=[
            pltpu.VMEM((SUPR, IDXW), jnp.int32),
            pltpu.VMEM((IDXW,), jnp.float32),
            pltpu.VMEM_SHARED((ANODES1,), jnp.float32),
        ],
        mesh=_mesh(),
        compiler_params=pltpu.CompilerParams(use_tc_tiling_on_sc=False),
    )
    def deg_kernel(dst_ref, z_ref, o0_ref, o1_ref, idx_v, ones_v, acc):
        c = lax.axis_index("c")
        s = lax.axis_index("s")

        pltpu.sync_copy(z_ref.at[pl.ds(s * DEG_CH, DEG_CH)],
                        acc.at[pl.ds(s * DEG_CH, DEG_CH)])
        for i in range(IDXW // 16):
            ones_v[pl.ds(i * 16, 16)] = jnp.ones((16,), jnp.float32)
        plsc.subcore_barrier()

        base = c * (NROWS // 2) + s * (NROWS // 32)

        def body(sup, carry):
            rb = base + sup * SUPR
            pltpu.sync_copy(dst_ref.at[pl.ds(rb, SUPR)], idx_v)
            for j in range(SUPR):
                pltpu.sync_copy(ones_v, acc.at[idx_v.at[j]], add=True)
            return carry

        lax.fori_loop(0, NROWS // 32 // SUPR, body, 0)
        plsc.subcore_barrier()

        @pl.when(c == 0)
        def _():
            pltpu.sync_copy(acc.at[pl.ds(s * DEG_CH, DEG_CH)],
                            o0_ref.at[pl.ds(s * DEG_CH, DEG_CH)])

        @pl.when(c == 1)
        def _():
            pltpu.sync_copy(acc.at[pl.ds(s * DEG_CH, DEG_CH)],
                            o1_ref.at[pl.ds(s * DEG_CH, DEG_CH)])

    return deg_kernel(dstp, zeros1d)


def _sc_edge_sum(srcs, dstp, ytab, zeros2d, feature_split):
    nsup = (NROWS // 16 if feature_split else NROWS // 32) // SUPR

    @functools.partial(
        pl.kernel,
        out_type=jax.ShapeDtypeStruct((2, N, 16), jnp.float32),
        scratch_types=[
            pltpu.VMEM((SUPR, IDXW), jnp.int32),
            pltpu.VMEM((SUPR, IDXW), jnp.int32),
            pltpu.VMEM((SUPR, IDXW, 16), jnp.float32),
            pltpu.VMEM_SHARED((ANODES, 16), jnp.float32),
            pltpu.SemaphoreType.DMA,
        ],
        mesh=_mesh(),
        compiler_params=pltpu.CompilerParams(use_tc_tiling_on_sc=False),
    )
    def conv_kernel(src_ref, dst_ref, tab_ref, z_ref, out_ref,
                    idxs_v, idxd_v, rows_v, acc, sem):
        c = lax.axis_index("c")
        s = lax.axis_index("s")

        @pl.when(s < 15)
        def _():
            pltpu.sync_copy(z_ref.at[pl.ds(s * ACC_CH, ACC_CH)],
                            acc.at[pl.ds(s * ACC_CH, ACC_CH)])

        @pl.when(s == 15)
        def _():
            pltpu.sync_copy(z_ref.at[pl.ds(15 * ACC_CH, ACC_LAST)],
                            acc.at[pl.ds(15 * ACC_CH, ACC_LAST)])

        plsc.subcore_barrier()

        if feature_split:
            base = s * (NROWS // 16)
        else:
            base = c * (NROWS // 2) + s * (NROWS // 32)

        def body(sup, carry):
            rb = base + sup * SUPR
            if feature_split:
                pltpu.sync_copy(src_ref.at[c, pl.ds(rb, SUPR)], idxs_v)
            else:
                pltpu.sync_copy(src_ref.at[0, pl.ds(rb, SUPR)], idxs_v)
            pltpu.sync_copy(dst_ref.at[pl.ds(rb, SUPR)], idxd_v)
            descs = [
                pltpu.async_copy(tab_ref.at[idxs_v.at[j]], rows_v.at[j], sem)
                for j in range(SUPR)
            ]
            for d in descs:
                d.wait()
            for j in range(SUPR):
                pltpu.sync_copy(rows_v.at[j], acc.at[idxd_v.at[j]], add=True)
            return carry

        lax.fori_loop(0, nsup, body, 0)
        plsc.subcore_barrier()

        @pl.when(s < 15)
        def _():
            pltpu.sync_copy(acc.at[pl.ds(s * ACC_CH, ACC_CH)],
                            out_ref.at[c, pl.ds(s * ACC_CH, ACC_CH)])

        @pl.when(s == 15)
        def _():
            pltpu.sync_copy(acc.at[pl.ds(15 * ACC_CH, ACC_LAST)],
                            out_ref.at[c, pl.ds(15 * ACC_CH, ACC_LAST)])

    return conv_kernel(srcs, dstp, ytab, zeros2d)


def _elu(z):
    return jnp.where(z > 0, z, jnp.exp(jnp.minimum(z, 0.0)) - 1.0)


def _full(shape):
    return pl.BlockSpec(shape, lambda i: tuple(0 for _ in shape))


def _rows(cols):
    return pl.BlockSpec((RB, cols), lambda i: (i, 0))


def _halves():
    return pl.BlockSpec((2, RB, 16), lambda i: (0, i, 0))


_DEG = pl.BlockSpec((RB, 2), lambda i: (i, 0))


def _accumulate(i, acc_ref, st_ref, *stats):
    @pl.when(i == 0)
    def _():
        acc_ref[...] = jnp.zeros_like(acc_ref)

    acc_ref[...] = acc_ref[...] + jnp.stack(stats).reshape(acc_ref.shape)

    @pl.when(i == GRID - 1)
    def _():
        st_ref[...] = acc_ref[...]


def _k_linear(xb, w, b, kin, kout):
    def body(x_ref, w_ref, b_ref, z_ref, st_ref, acc_ref):
        i = pl.program_id(0)
        z = jnp.dot(x_ref[...], w_ref[...],
                    preferred_element_type=jnp.float32) + b_ref[...]
        z_ref[...] = z
        _accumulate(i, acc_ref, st_ref, jnp.sum(z, 0), jnp.sum(z * z, 0))

    return pl.pallas_call(
        body,
        grid=(GRID,),
        in_specs=[_rows(kin), _full((kin, kout)), _full((1, kout))],
        out_specs=[_rows(kout), _full((2, kout))],
        out_shape=[jax.ShapeDtypeStruct((N, kout), jnp.float32),
                   jax.ShapeDtypeStruct((2, kout), jnp.float32)],
        scratch_shapes=[pltpu.VMEM((2, kout), jnp.float32)],
    )(xb, w, b)


def _k_norm_linear(zb, sc, sh, w, b, kin, kout):
    def body(z_ref, sc_ref, sh_ref, w_ref, b_ref, z2_ref, st_ref, acc_ref):
        i = pl.program_id(0)
        a = _elu(z_ref[...] * sc_ref[...] + sh_ref[...])
        z2 = jnp.dot(a, w_ref[...], preferred_element_type=jnp.float32) + b_ref[...]
        z2_ref[...] = z2
        _accumulate(i, acc_ref, st_ref, jnp.sum(z2, 0), jnp.sum(z2 * z2, 0))

    return pl.pallas_call(
        body,
        grid=(GRID,),
        in_specs=[_rows(kin), _full((1, kin)), _full((1, kin)),
                  _full((kin, kout)), _full((1, kout))],
        out_specs=[_rows(kout), _full((2, kout))],
        out_shape=[jax.ShapeDtypeStruct((N, kout), jnp.float32),
                   jax.ShapeDtypeStruct((2, kout), jnp.float32)],
        scratch_shapes=[pltpu.VMEM((2, kout), jnp.float32)],
    )(zb, sc, sh, w, b)


def _k_ytab1(z2, sc, sh, wg1r, deg3d):
    def body(z_ref, sc_ref, sh_ref, w_ref, d_ref, y_ref):
        h2 = _elu(z_ref[...] * sc_ref[...] + sh_ref[...])
        deg = d_ref[:, 0] + d_ref[:, 1] + 1.0
        dinv = lax.rsqrt(deg)[:, None]
        y_ref[0] = dinv * jnp.dot(h2, w_ref[0], preferred_element_type=jnp.float32)
        y_ref[1] = dinv * jnp.dot(h2, w_ref[1], preferred_element_type=jnp.float32)

    return pl.pallas_call(
        body,
        grid=(GRID,),
        in_specs=[_rows(32), _full((1, 32)), _full((1, 32)),
                  _full((2, 32, 16)), _DEG],
        out_specs=_halves(),
        out_shape=jax.ShapeDtypeStruct((2, N, 16), jnp.float32),
    )(z2, sc, sh, wg1r, deg3d)


def _k_combine1(acc1, ytab1, deg3d, bg1h):
    def body(a_ref, y_ref, d_ref, b_ref, u_ref, st_ref, acc_ref):
        i = pl.program_id(0)
        deg = d_ref[:, 0] + d_ref[:, 1] + 1.0
        dinv = lax.rsqrt(deg)[:, None]
        u0 = dinv * (y_ref[0] + a_ref[0]) + b_ref[0]
        u1 = dinv * (y_ref[1] + a_ref[1]) + b_ref[1]
        u_ref[0] = u0
        u_ref[1] = u1
        _accumulate(i, acc_ref, st_ref,
                    jnp.sum(u0, 0), jnp.sum(u1, 0),
                    jnp.sum(u0 * u0, 0), jnp.sum(u1 * u1, 0))

    return pl.pallas_call(
        body,
        grid=(GRID,),
        in_specs=[_halves(), _halves(), _DEG, _full((2, 1, 16))],
        out_specs=[_halves(), _full((2, 2, 16))],
        out_shape=[jax.ShapeDtypeStruct((2, N, 16), jnp.float32),
                   jax.ShapeDtypeStruct((2, 2, 16), jnp.float32)],
        scratch_shapes=[pltpu.VMEM((2, 2, 16), jnp.float32)],
    )(acc1, ytab1, deg3d, bg1h)


def _k_y2(u, sc3h, sh3h, wg2r, deg3d):
    def body(u_ref, sc_ref, sh_ref, w_ref, d_ref, y_ref):
        h0 = jnp.maximum(u_ref[0] * sc_ref[0] + sh_ref[0], 0.0)
        h1 = jnp.maximum(u_ref[1] * sc_ref[1] + sh_ref[1], 0.0)
        xw = (jnp.dot(h0, w_ref[0], preferred_element_type=jnp.float32)
              + jnp.dot(h1, w_ref[1], preferred_element_type=jnp.float32))
        deg = d_ref[:, 0] + d_ref[:, 1] + 1.0
        y_ref[...] = lax.rsqrt(deg)[:, None] * xw

    return pl.pallas_call(
        body,
        grid=(GRID,),
        in_specs=[_halves(), _full((2, 1, 16)), _full((2, 1, 16)),
                  _full((2, 16, 16)), _DEG],
        out_specs=_rows(16),
        out_shape=jax.ShapeDtypeStruct((N, 16), jnp.float32),
    )(u, sc3h, sh3h, wg2r, deg3d)


def _k_combine2(acc2, y2, deg3d, bg2):
    def body(a_ref, y_ref, d_ref, b_ref, v_ref, st_ref, acc_ref):
        i = pl.program_id(0)
        deg = d_ref[:, 0] + d_ref[:, 1] + 1.0
        dinv = lax.rsqrt(deg)[:, None]
        v = dinv * (y_ref[...] + a_ref[0] + a_ref[1]) + b_ref[...]
        v_ref[...] = v
        _accumulate(i, acc_ref, st_ref, jnp.sum(v, 0), jnp.sum(v * v, 0))

    return pl.pallas_call(
        body,
        grid=(GRID,),
        in_specs=[_halves(), _rows(16), _DEG, _full((1, 16))],
        out_specs=[_rows(16), _full((2, 16))],
        out_shape=[jax.ShapeDtypeStruct((N, 16), jnp.float32),
                   jax.ShapeDtypeStruct((2, 16), jnp.float32)],
        scratch_shapes=[pltpu.VMEM((2, 16), jnp.float32)],
    )(acc2, y2, deg3d, bg2)


def _k_norm(v, sc, sh):
    def body(v_ref, sc_ref, sh_ref, o_ref):
        o_ref[...] = v_ref[...] * sc_ref[...] + sh_ref[...]

    return pl.pallas_call(
        body,
        grid=(GRID,),
        in_specs=[_rows(16), _full((1, 16)), _full((1, 16))],
        out_specs=_rows(16),
        out_shape=jax.ShapeDtypeStruct((N, 16), jnp.float32),
    )(v, sc, sh)


def _bn_fold(st, gamma, beta, eps):
    mu = st[0] / N
    var = st[1] / N - mu * mu
    scale = gamma / jnp.sqrt(var + eps)
    shift = beta - mu * scale
    return scale, shift


def kernel(x, edge_index, W1, b1, g1, be1, W2, b2, g2, be2,
           Wg1, bg1, gg1, gb1, Wg2, bg2, gg2, gb2):
    src = edge_index[0].astype(jnp.int32)
    dst = edge_index[1].astype(jnp.int32)
    src0 = jnp.concatenate([src, jnp.zeros((PADE,), jnp.int32)])
    srcs = jnp.stack([src0, src0 + N]).reshape(2, NROWS, IDXW)
    dstp = jnp.concatenate(
        [dst, jnp.full((PADE,), N, jnp.int32)]).reshape(NROWS, IDXW)
    zeros1d = jnp.zeros((ANODES1,), jnp.float32)
    zeros2d = jnp.zeros((N, 16), jnp.float32)

    d0, d1 = _sc_degree(dstp, zeros1d)
    deg3d = jnp.stack([d0[:N], d1[:N]], axis=1)

    z1, st1 = _k_linear(x, W1, b1.reshape(1, 64), 128, 64)
    sc1, sh1 = _bn_fold(st1, g1, be1, 1e-3)
    z2, st2 = _k_norm_linear(z1, sc1.reshape(1, 64), sh1.reshape(1, 64),
                             W2, b2.reshape(1, 32), 64, 32)
    sc2, sh2 = _bn_fold(st2, g2, be2, 1e-3)

    ytab1 = _k_ytab1(z2, sc2.reshape(1, 32), sh2.reshape(1, 32),
                     Wg1.reshape(32, 2, 16).transpose(1, 0, 2), deg3d)
    acc1 = _sc_edge_sum(srcs, dstp, ytab1.reshape(2 * N, 16), zeros2d,
                        feature_split=True)
    u, st3h = _k_combine1(acc1, ytab1, deg3d, bg1.reshape(2, 1, 16))
    st3 = st3h.reshape(2, 32)
    sc3, sh3 = _bn_fold(st3, gg1, gb1, 1e-5)

    y2 = _k_y2(u, sc3.reshape(2, 1, 16), sh3.reshape(2, 1, 16),
               Wg2.reshape(2, 16, 16), deg3d)
    acc2 = _sc_edge_sum(srcs, dstp, y2, zeros2d, feature_split=False)
    v, st4 = _k_combine2(acc2, y2, deg3d, bg2.reshape(1, 16))
    sc4, sh4 = _bn_fold(st4, gg2, gb2, 1e-5)
    return _k_norm(v, sc4.reshape(1, 16), sh4.reshape(1, 16))


# speedup vs baseline: 32.6763x; 1.1986x over previous
import functools

import jax
import jax.numpy as jnp
from jax import lax
from jax.experimental import pallas as pl
from jax.experimental.pallas import tpu as pltpu
from jax.experimental.pallas import tpu_sc as plsc

N = 100000
E = 1600000
IDXW = 128
SUPR = 4
EPAD = 1605632
NROWS = EPAD // IDXW
PADE = EPAD - E
ANODES = N + 8
RB = 5000
GRID = N // RB

def _mesh():
    return plsc.VectorSubcoreMesh(core_axis_name="c", subcore_axis_name="s")

ANODES1 = 16 * 6272
DEG_CH = 6272
ACC_CH = 6256
ACC_LAST = N - 15 * ACC_CH


def _sc_degree(idxp, zeros1d):
    @functools.partial(
        pl.kernel,
        out_type=(jax.ShapeDtypeStruct((ANODES1,), jnp.float32),
                  jax.ShapeDtypeStruct((ANODES1,), jnp.float32)),
        scratch_types=[
            pltpu.VMEM((SUPR, IDXW), jnp.int32),
            pltpu.VMEM((SUPR, IDXW), jnp.int32),
            pltpu.VMEM((IDXW,), jnp.float32),
            pltpu.VMEM_SHARED((ANODES1,), jnp.float32),
            pltpu.SemaphoreType.DMA,
            pltpu.SemaphoreType.DMA,
        ],
        mesh=_mesh(),
        compiler_params=pltpu.CompilerParams(use_tc_tiling_on_sc=False),
    )
    def deg_kernel(idx_ref, z_ref, o0_ref, o1_ref, idx0_v, idx1_v, ones_v,
                   acc, ssem0, ssem1):
        c = lax.axis_index("c")
        s = lax.axis_index("s")

        pltpu.sync_copy(z_ref.at[pl.ds(s * DEG_CH, DEG_CH)],
                        acc.at[pl.ds(s * DEG_CH, DEG_CH)])
        for i in range(IDXW // 16):
            ones_v[pl.ds(i * 16, 16)] = jnp.ones((16,), jnp.float32)
        plsc.subcore_barrier()

        base = c * (NROWS // 2) + s * (NROWS // 32)
        bufs = ((idx0_v, ssem0), (idx1_v, ssem1))

        def scat(idx_v, ssem, j):
            return pltpu.make_async_copy(ones_v, acc.at[idx_v.at[j]], ssem)

        def body(g, carry):
            for b, (idx_v, ssem) in enumerate(bufs):
                @pl.when(g > 0)
                def _():
                    for j in range(SUPR):
                        scat(idx_v, ssem, j).wait()

                rb = base + (2 * g + b) * SUPR
                pltpu.sync_copy(idx_ref.at[2, pl.ds(rb, SUPR)], idx_v)
                for j in range(SUPR):
                    scat(idx_v, ssem, j).start(add=True)
            return carry

        lax.fori_loop(0, NROWS // 32 // SUPR // 2, body, 0)
        for idx_v, ssem in bufs:
            for j in range(SUPR):
                scat(idx_v, ssem, j).wait()
        plsc.subcore_barrier()

        @pl.when(c == 0)
        def _():
            pltpu.sync_copy(acc.at[pl.ds(s * DEG_CH, DEG_CH)],
                            o0_ref.at[pl.ds(s * DEG_CH, DEG_CH)])

        @pl.when(c == 1)
        def _():
            pltpu.sync_copy(acc.at[pl.ds(s * DEG_CH, DEG_CH)],
                            o1_ref.at[pl.ds(s * DEG_CH, DEG_CH)])

    return deg_kernel(idxp, zeros1d)


def _sc_edge_sum(idxp, ytab, zeros2d, feature_split):
    npairs = (NROWS // 16 if feature_split else NROWS // 32) // SUPR // 2

    @functools.partial(
        pl.kernel,
        out_type=jax.ShapeDtypeStruct((2, N, 16), jnp.float32),
        scratch_types=[
            pltpu.VMEM((SUPR, IDXW), jnp.int32),
            pltpu.VMEM((SUPR, IDXW), jnp.int32),
            pltpu.VMEM((SUPR, IDXW), jnp.int32),
            pltpu.VMEM((SUPR, IDXW), jnp.int32),
            pltpu.VMEM((SUPR, IDXW, 16), jnp.float32),
            pltpu.VMEM((SUPR, IDXW, 16), jnp.float32),
            pltpu.VMEM_SHARED((ANODES, 16), jnp.float32),
            pltpu.SemaphoreType.DMA,
            pltpu.SemaphoreType.DMA,
            pltpu.SemaphoreType.DMA,
            pltpu.SemaphoreType.DMA,
        ],
        mesh=_mesh(),
        compiler_params=pltpu.CompilerParams(use_tc_tiling_on_sc=False),
    )
    def conv_kernel(idx_ref, tab_ref, z_ref, out_ref,
                    src0_v, src1_v, dst0_v, dst1_v, rows0_v, rows1_v, acc,
                    gsem0, gsem1, ssem0, ssem1):
        c = lax.axis_index("c")
        s = lax.axis_index("s")

        @pl.when(s < 15)
        def _():
            pltpu.sync_copy(z_ref.at[pl.ds(s * ACC_CH, ACC_CH)],
                            acc.at[pl.ds(s * ACC_CH, ACC_CH)])

        @pl.when(s == 15)
        def _():
            pltpu.sync_copy(z_ref.at[pl.ds(15 * ACC_CH, ACC_LAST)],
                            acc.at[pl.ds(15 * ACC_CH, ACC_LAST)])

        plsc.subcore_barrier()

        if feature_split:
            base = s * (NROWS // 16)
        else:
            base = c * (NROWS // 2) + s * (NROWS // 32)

        bufs = ((src0_v, dst0_v, rows0_v, gsem0, ssem0),
                (src1_v, dst1_v, rows1_v, gsem1, ssem1))

        def scat(dst_v, rows_v, ssem, j):
            return pltpu.make_async_copy(
                rows_v.at[j], acc.at[dst_v.at[j]], ssem)

        def body(g, carry):
            gd = []
            for b, (src_v, dst_v, rows_v, gsem, ssem) in enumerate(bufs):
                @pl.when(g > 0)
                def _():
                    for j in range(SUPR):
                        scat(dst_v, rows_v, ssem, j).wait()

                rb = base + (2 * g + b) * SUPR
                if feature_split:
                    pltpu.sync_copy(idx_ref.at[c, pl.ds(rb, SUPR)], src_v)
                else:
                    pltpu.sync_copy(idx_ref.at[0, pl.ds(rb, SUPR)], src_v)
                pltpu.sync_copy(idx_ref.at[2, pl.ds(rb, SUPR)], dst_v)
                gd.append([
                    pltpu.async_copy(tab_ref.at[src_v.at[j]],
                                     rows_v.at[j], gsem)
                    for j in range(SUPR)
                ])
            for b, (src_v, dst_v, rows_v, gsem, ssem) in enumerate(bufs):
                for d in gd[b]:
                    d.wait()
                for j in range(SUPR):
                    scat(dst_v, rows_v, ssem, j).start(add=True)
            return carry

        lax.fori_loop(0, npairs, body, 0)
        for src_v, dst_v, rows_v, gsem, ssem in bufs:
            for j in range(SUPR):
                scat(dst_v, rows_v, ssem, j).wait()
        plsc.subcore_barrier()

        @pl.when(s < 15)
        def _():
            pltpu.sync_copy(acc.at[pl.ds(s * ACC_CH, ACC_CH)],
                            out_ref.at[c, pl.ds(s * ACC_CH, ACC_CH)])

        @pl.when(s == 15)
        def _():
            pltpu.sync_copy(acc.at[pl.ds(15 * ACC_CH, ACC_LAST)],
                            out_ref.at[c, pl.ds(15 * ACC_CH, ACC_LAST)])

    return conv_kernel(idxp, ytab, zeros2d)


def _elu(z):
    return jnp.where(z > 0, z, jnp.exp(jnp.minimum(z, 0.0)) - 1.0)


def _full(shape):
    return pl.BlockSpec(shape, lambda i: tuple(0 for _ in shape))


def _rows(cols):
    return pl.BlockSpec((RB, cols), lambda i: (i, 0))


def _halves():
    return pl.BlockSpec((2, RB, 16), lambda i: (0, i, 0))


_DEG = pl.BlockSpec((1, 1, RB), lambda i: (i, 0, 0))


def _dinv_of(d0_ref, d1_ref, i):
    del i
    deg = d0_ref[0, 0] + d1_ref[0, 0] + 1.0
    return lax.rsqrt(deg)[:, None]


def _accumulate(i, acc_ref, st_ref, *stats):
    @pl.when(i == 0)
    def _():
        acc_ref[...] = jnp.zeros_like(acc_ref)

    acc_ref[...] = acc_ref[...] + jnp.stack(stats).reshape(acc_ref.shape)

    @pl.when(i == GRID - 1)
    def _():
        st_ref[...] = acc_ref[...]


def _k_linear(xb, w, b, kin, kout):
    def body(x_ref, w_ref, b_ref, z_ref, st_ref, acc_ref):
        i = pl.program_id(0)
        z = jnp.dot(x_ref[...], w_ref[...],
                    preferred_element_type=jnp.float32) + b_ref[...]
        z_ref[...] = z
        _accumulate(i, acc_ref, st_ref, jnp.sum(z, 0), jnp.sum(z * z, 0))

    return pl.pallas_call(
        body,
        grid=(GRID,),
        in_specs=[_rows(kin), _full((kin, kout)), _full((1, kout))],
        out_specs=[_rows(kout), _full((2, kout))],
        out_shape=[jax.ShapeDtypeStruct((N, kout), jnp.float32),
                   jax.ShapeDtypeStruct((2, kout), jnp.float32)],
        scratch_shapes=[pltpu.VMEM((2, kout), jnp.float32)],
    )(xb, w, b)


def _k_norm_linear(zb, sc, sh, w, b, kin, kout):
    def body(z_ref, sc_ref, sh_ref, w_ref, b_ref, z2_ref, st_ref, acc_ref):
        i = pl.program_id(0)
        a = _elu(z_ref[...] * sc_ref[...] + sh_ref[...])
        z2 = jnp.dot(a, w_ref[...], preferred_element_type=jnp.float32) + b_ref[...]
        z2_ref[...] = z2
        _accumulate(i, acc_ref, st_ref, jnp.sum(z2, 0), jnp.sum(z2 * z2, 0))

    return pl.pallas_call(
        body,
        grid=(GRID,),
        in_specs=[_rows(kin), _full((1, kin)), _full((1, kin)),
                  _full((kin, kout)), _full((1, kout))],
        out_specs=[_rows(kout), _full((2, kout))],
        out_shape=[jax.ShapeDtypeStruct((N, kout), jnp.float32),
                   jax.ShapeDtypeStruct((2, kout), jnp.float32)],
        scratch_shapes=[pltpu.VMEM((2, kout), jnp.float32)],
    )(zb, sc, sh, w, b)


def _k_ytab1(z2, sc, sh, wg1r, d0, d1):
    def body(z_ref, sc_ref, sh_ref, w_ref, d0_ref, d1_ref, y_ref):
        h2 = _elu(z_ref[...] * sc_ref[...] + sh_ref[...])
        dinv = _dinv_of(d0_ref, d1_ref, pl.program_id(0))
        y_ref[0] = dinv * jnp.dot(h2, w_ref[0], preferred_element_type=jnp.float32)
        y_ref[1] = dinv * jnp.dot(h2, w_ref[1], preferred_element_type=jnp.float32)

    return pl.pallas_call(
        body,
        grid=(GRID,),
        in_specs=[_rows(32), _full((1, 32)), _full((1, 32)),
                  _full((2, 32, 16)), _DEG, _DEG],
        out_specs=_halves(),
        out_shape=jax.ShapeDtypeStruct((2, N, 16), jnp.float32),
    )(z2, sc, sh, wg1r, d0, d1)


def _k_combine1(acc1, ytab1, d0, d1, bg1h):
    def body(a_ref, y_ref, d0_ref, d1_ref, b_ref, u_ref, st_ref, acc_ref):
        i = pl.program_id(0)
        dinv = _dinv_of(d0_ref, d1_ref, i)
        u0 = dinv * (y_ref[0] + a_ref[0]) + b_ref[0]
        u1 = dinv * (y_ref[1] + a_ref[1]) + b_ref[1]
        u_ref[0] = u0
        u_ref[1] = u1
        _accumulate(i, acc_ref, st_ref,
                    jnp.sum(u0, 0), jnp.sum(u1, 0),
                    jnp.sum(u0 * u0, 0), jnp.sum(u1 * u1, 0))

    return pl.pallas_call(
        body,
        grid=(GRID,),
        in_specs=[_halves(), _halves(), _DEG, _DEG, _full((2, 1, 16))],
        out_specs=[_halves(), _full((2, 2, 16))],
        out_shape=[jax.ShapeDtypeStruct((2, N, 16), jnp.float32),
                   jax.ShapeDtypeStruct((2, 2, 16), jnp.float32)],
        scratch_shapes=[pltpu.VMEM((2, 2, 16), jnp.float32)],
    )(acc1, ytab1, d0, d1, bg1h)


def _k_y2(u, sc3h, sh3h, wg2r, d0, d1):
    def body(u_ref, sc_ref, sh_ref, w_ref, d0_ref, d1_ref, y_ref):
        h0 = jnp.maximum(u_ref[0] * sc_ref[0] + sh_ref[0], 0.0)
        h1 = jnp.maximum(u_ref[1] * sc_ref[1] + sh_ref[1], 0.0)
        xw = (jnp.dot(h0, w_ref[0], preferred_element_type=jnp.float32)
              + jnp.dot(h1, w_ref[1], preferred_element_type=jnp.float32))
        y_ref[...] = _dinv_of(d0_ref, d1_ref, pl.program_id(0)) * xw

    return pl.pallas_call(
        body,
        grid=(GRID,),
        in_specs=[_halves(), _full((2, 1, 16)), _full((2, 1, 16)),
                  _full((2, 16, 16)), _DEG, _DEG],
        out_specs=_rows(16),
        out_shape=jax.ShapeDtypeStruct((N, 16), jnp.float32),
    )(u, sc3h, sh3h, wg2r, d0, d1)


def _k_combine2(acc2, y2, d0, d1, bg2):
    def body(a_ref, y_ref, d0_ref, d1_ref, b_ref, v_ref, st_ref, acc_ref):
        i = pl.program_id(0)
        dinv = _dinv_of(d0_ref, d1_ref, i)
        v = dinv * (y_ref[...] + a_ref[0] + a_ref[1]) + b_ref[...]
        v_ref[...] = v
        _accumulate(i, acc_ref, st_ref, jnp.sum(v, 0), jnp.sum(v * v, 0))

    return pl.pallas_call(
        body,
        grid=(GRID,),
        in_specs=[_halves(), _rows(16), _DEG, _DEG, _full((1, 16))],
        out_specs=[_rows(16), _full((2, 16))],
        out_shape=[jax.ShapeDtypeStruct((N, 16), jnp.float32),
                   jax.ShapeDtypeStruct((2, 16), jnp.float32)],
        scratch_shapes=[pltpu.VMEM((2, 16), jnp.float32)],
    )(acc2, y2, d0, d1, bg2)


def _k_norm(v, sc, sh):
    def body(v_ref, sc_ref, sh_ref, o_ref):
        o_ref[...] = v_ref[...] * sc_ref[...] + sh_ref[...]

    return pl.pallas_call(
        body,
        grid=(GRID,),
        in_specs=[_rows(16), _full((1, 16)), _full((1, 16))],
        out_specs=_rows(16),
        out_shape=jax.ShapeDtypeStruct((N, 16), jnp.float32),
    )(v, sc, sh)


def _bn_fold(st, gamma, beta, eps):
    mu = st[0] / N
    var = st[1] / N - mu * mu
    scale = gamma / jnp.sqrt(var + eps)
    shift = beta - mu * scale
    return scale, shift


def kernel(x, edge_index, W1, b1, g1, be1, W2, b2, g2, be2,
           Wg1, bg1, gg1, gb1, Wg2, bg2, gg2, gb2):
    src = edge_index[0].astype(jnp.int32)
    dst = edge_index[1].astype(jnp.int32)
    srcp = jnp.concatenate([src, jnp.zeros((PADE,), jnp.int32)])
    idxp = jnp.stack([
        srcp,
        srcp + N,
        jnp.concatenate([dst, jnp.full((PADE,), N, jnp.int32)]),
    ]).reshape(3, NROWS, IDXW)
    zeros1d = jnp.zeros((ANODES1,), jnp.float32)
    zeros2d = jnp.zeros((N, 16), jnp.float32)

    d0, d1 = _sc_degree(idxp, zeros1d)
    d0 = d0[:N].reshape(GRID, 1, RB)
    d1 = d1[:N].reshape(GRID, 1, RB)

    z1, st1 = _k_linear(x, W1, b1.reshape(1, 64), 128, 64)
    sc1, sh1 = _bn_fold(st1, g1, be1, 1e-3)
    z2, st2 = _k_norm_linear(z1, sc1.reshape(1, 64), sh1.reshape(1, 64),
                             W2, b2.reshape(1, 32), 64, 32)
    sc2, sh2 = _bn_fold(st2, g2, be2, 1e-3)

    ytab1 = _k_ytab1(z2, sc2.reshape(1, 32), sh2.reshape(1, 32),
                     Wg1.reshape(32, 2, 16).transpose(1, 0, 2), d0, d1)
    acc1 = _sc_edge_sum(idxp, ytab1.reshape(2 * N, 16), zeros2d,
                        feature_split=True)
    u, st3h = _k_combine1(acc1, ytab1, d0, d1, bg1.reshape(2, 1, 16))
    st3 = st3h.reshape(2, 32)
    sc3, sh3 = _bn_fold(st3, gg1, gb1, 1e-5)

    y2 = _k_y2(u, sc3.reshape(2, 1, 16), sh3.reshape(2, 1, 16),
               Wg2.reshape(2, 16, 16), d0, d1)
    acc2 = _sc_edge_sum(idxp, y2, zeros2d, feature_split=False)
    v, st4 = _k_combine2(acc2, y2, d0, d1, bg2.reshape(1, 16))
    sc4, sh4 = _bn_fold(st4, gg2, gb2, 1e-5)
    return _k_norm(v, sc4.reshape(1, 16), sh4.reshape(1, 16))
